# Initial kernel scaffold; baseline (speedup 1.0000x reference)
import jax
import jax.numpy as jnp
from jax.experimental import pallas as pl


def kernel(pred_logits, center_mask):
    raise NotImplementedError("write your pallas kernel here")


# speedup vs baseline: 3.5455x; 3.5455x over previous
import functools

import jax
import jax.numpy as jnp
from jax import lax
from jax.experimental import pallas as pl
from jax.experimental.pallas import tpu as pltpu
from jax.experimental.pallas import tpu_sc as plsc

NEG = -1e30
BZ, NQ, NC = 8, 5000, 1001
NV = NC - 1
QB = 1000
J = NQ // QB
NL = 16
NQP = 5120
TBLP = 1008
TBL = NL * TBLP


def _stats_body(x_ref, madd_ref, idx_ref, sm_ref, vmax_ref, isum_ref):
    ql = x_ref[0] + madd_ref[0]
    lane = lax.broadcasted_iota(jnp.int32, (QB, NC), 1)
    full_max = jnp.max(ql, axis=-1)
    idx = jnp.min(jnp.where(ql == full_max[:, None], lane, NC), axis=-1)
    vmax = jnp.max(jnp.where(lane < NV, ql, NEG), axis=-1)
    bg = jnp.max(jnp.where(lane == NV, ql, NEG), axis=-1)
    e = jnp.exp(ql - vmax[:, None])
    sum_all = jnp.sum(e, axis=-1)
    sum_vec = sum_all - jnp.exp(bg - vmax)
    sum_full = jnp.exp(vmax - full_max) * sum_vec + jnp.exp(bg - full_max)
    score = 1.0 / sum_full
    valid = idx < NV
    sm = jnp.where(valid, score, NEG)
    idx_ref[0, 0] = idx
    sm_ref[0, 0] = sm
    vmax_ref[0, 0] = vmax
    isum_ref[0, 0] = 1.0 / sum_vec


def _run_stats(x, madd):
    ospec = pl.BlockSpec((1, 1, QB), lambda b, j: (b * J + j, 0, 0))
    st = jax.ShapeDtypeStruct((BZ * J, 1, QB), jnp.float32)
    return pl.pallas_call(
        _stats_body,
        grid=(BZ, J),
        in_specs=[
            pl.BlockSpec((1, QB, NC), lambda b, j: (b, j, 0)),
            pl.BlockSpec((1, 1, NC), lambda b, j: (b, 0, 0)),
        ],
        out_specs=(ospec, ospec, ospec, ospec),
        out_shape=(jax.ShapeDtypeStruct((BZ * J, 1, QB), jnp.int32), st, st, st),
        compiler_params=pltpu.CompilerParams(
            dimension_semantics=("parallel", "parallel")),
    )(x, madd)


def _dedup_body(idx_hbm, sm_hbm, isum_hbm, out_hbm,
                idx_v, sm_v, isum_v, scale_v, tbl_v):
    wid = lax.axis_index("s") * 2 + lax.axis_index("c")

    @pl.when(wid < BZ)
    def _():
        b = wid
        pltpu.sync_copy(idx_hbm.at[b], idx_v)
        pltpu.sync_copy(sm_hbm.at[b], sm_v)
        pltpu.sync_copy(isum_hbm.at[b], isum_v)

        def init(i, c):
            tbl_v[pl.ds(i * NL, NL)] = jnp.full((NL,), NEG, jnp.float32)
            return c
        lax.fori_loop(0, TBL // NL, init, 0)

        lane_base = lax.iota(jnp.int32, NL) * TBLP

        def smax(i, c):
            idx16 = idx_v[pl.ds(i * NL, NL)]
            s16 = sm_v[pl.ds(i * NL, NL)]
            addr = lane_base + idx16
            t = plsc.load_gather(tbl_v, [addr])
            plsc.store_scatter(tbl_v, [addr], jnp.maximum(t, s16))
            return c
        lax.fori_loop(0, NQP // NL, smax, 0)

        def merge(i, c):
            off = i * NL
            acc = tbl_v[pl.ds(off, NL)]
            for l in range(1, NL):
                acc = jnp.maximum(acc, tbl_v[pl.ds(l * TBLP + off, NL)])
            tbl_v[pl.ds(off, NL)] = acc
            return c
        lax.fori_loop(0, TBLP // NL, merge, 0)

        def emit(i, c):
            idx16 = idx_v[pl.ds(i * NL, NL)]
            s16 = sm_v[pl.ds(i * NL, NL)]
            t = plsc.load_gather(tbl_v, [idx16])
            keep = jnp.logical_and(s16 == t, idx16 < NV)
            scale_v[pl.ds(i * NL, NL)] = jnp.where(
                keep, isum_v[pl.ds(i * NL, NL)], jnp.float32(0.0))
            return c
        lax.fori_loop(0, NQP // NL, emit, 0)

        pltpu.sync_copy(scale_v, out_hbm.at[b])


@functools.lru_cache(maxsize=1)
def _make_dedup():
    return pl.kernel(
        _dedup_body,
        mesh=plsc.VectorSubcoreMesh(core_axis_name="c", subcore_axis_name="s"),
        compiler_params=pltpu.CompilerParams(needs_layout_passes=False),
        out_type=jax.ShapeDtypeStruct((BZ, NQP), jnp.float32),
        scratch_types=[
            pltpu.VMEM((NQP,), jnp.int32),
            pltpu.VMEM((NQP,), jnp.float32),
            pltpu.VMEM((NQP,), jnp.float32),
            pltpu.VMEM((NQP,), jnp.float32),
            pltpu.VMEM((TBL,), jnp.float32),
        ],
    )


def _out_body(x_ref, madd_ref, vmax_ref, scale_ref, o_ref):
    ql = x_ref[0] + madd_ref[0]
    vmax = vmax_ref[0, 0]
    scale = scale_ref[0, 0]
    e = jnp.exp(ql[:, :NV] - vmax[:, None])
    o_ref[0] = e * scale[:, None]


def _run_out(x, madd, vmax, scale):
    sspec = pl.BlockSpec((1, 1, QB), lambda b, j: (b * J + j, 0, 0))
    return pl.pallas_call(
        _out_body,
        grid=(BZ, J),
        in_specs=[
            pl.BlockSpec((1, QB, NC), lambda b, j: (b, j, 0)),
            pl.BlockSpec((1, 1, NC), lambda b, j: (b, 0, 0)),
            sspec,
            sspec,
        ],
        out_specs=pl.BlockSpec((1, QB, NV), lambda b, j: (b, j, 0)),
        out_shape=jax.ShapeDtypeStruct((BZ, NQ, NV), jnp.float32),
        compiler_params=pltpu.CompilerParams(
            dimension_semantics=("parallel", "parallel")),
    )(x, madd, vmax, scale)


def kernel(pred_logits, center_mask):
    qmask = jnp.concatenate(
        [center_mask, jnp.ones((BZ, 1), dtype=bool)], axis=1)
    madd = jnp.where(qmask, jnp.float32(0.0), NEG)[:, None, :]

    idx, sm, vmax, isum = _run_stats(pred_logits, madd)

    pad = NQP - NQ
    idx_p = jnp.pad(idx.reshape(BZ, NQ), ((0, 0), (0, pad)),
                    constant_values=NV)
    sm_p = jnp.pad(sm.reshape(BZ, NQ), ((0, 0), (0, pad)),
                   constant_values=-1e30)
    isum_p = jnp.pad(isum.reshape(BZ, NQ), ((0, 0), (0, pad)))

    scale_p = _make_dedup()(idx_p, sm_p, isum_p)
    scale = scale_p[:, :NQ].reshape(BZ * J, 1, QB)

    return _run_out(pred_logits, madd, vmax, scale)
